# Initial kernel scaffold; baseline (speedup 1.0000x reference)
import jax
import jax.numpy as jnp
from jax.experimental import pallas as pl


def kernel(x, offsets, fc_weight, bias):
    raise NotImplementedError("write your pallas kernel here")


# speedup vs baseline: 1.1396x; 1.1396x over previous
import functools

import jax
import jax.numpy as jnp
from jax import lax
from jax.experimental import pallas as pl
from jax.experimental.pallas import tpu as pltpu
from jax.experimental.pallas import tpu_sc as plsc

BATCH = 16384
NUM_FIELDS = 26
TOTAL_ROWS = 100000 * 26

NUM_CORES = 2
NUM_SUBCORES = 16
LANES = 16
NUM_WORKERS = NUM_CORES * NUM_SUBCORES

B_PER_W = BATCH // NUM_WORKERS
IDX_PER_W = B_PER_W * NUM_FIELDS
IDX_CHUNKS = IDX_PER_W // LANES
RED_STEPS = B_PER_W // LANES


@functools.partial(
    pl.kernel,
    out_type=jax.ShapeDtypeStruct((BATCH,), jnp.float32),
    mesh=plsc.VectorSubcoreMesh(core_axis_name="c", subcore_axis_name="s"),
    compiler_params=pltpu.CompilerParams(needs_layout_passes=False),
    scratch_types=[
        pltpu.VMEM((IDX_PER_W,), jnp.int32),
        pltpu.VMEM((IDX_PER_W,), jnp.float32),
        pltpu.VMEM((NUM_FIELDS,), jnp.int32),
        pltpu.VMEM((1,), jnp.float32),
        pltpu.VMEM((B_PER_W,), jnp.float32),
        pltpu.SemaphoreType.DMA,
    ],
)
def _fl_kernel(x_hbm, offs_hbm, table_hbm, bias_hbm, out_hbm,
               idx_v, vals_v, offs_v, bias_v, out_v, sem):
    wid = lax.axis_index("s") * NUM_CORES + lax.axis_index("c")
    base = wid * B_PER_W

    pltpu.sync_copy(x_hbm.at[pl.ds(base * NUM_FIELDS, IDX_PER_W)], idx_v)
    pltpu.sync_copy(offs_hbm, offs_v)
    pltpu.sync_copy(bias_hbm, bias_v)

    iota16 = lax.iota(jnp.int32, LANES)

    def add_offsets(j, carry):
        pv = iota16 + j * LANES
        fv = lax.rem(pv, NUM_FIELDS)
        off = plsc.load_gather(offs_v, [fv])
        sl = pl.ds(j * LANES, LANES)
        idx_v[sl] = idx_v[sl] + off
        return carry

    lax.fori_loop(0, IDX_CHUNKS, add_offsets, 0)

    pltpu.async_copy(table_hbm.at[idx_v], vals_v, sem).wait()

    bias_vec = plsc.load_gather(bias_v, [jnp.zeros((LANES,), jnp.int32)])
    lane_off = iota16 * NUM_FIELDS

    def reduce_rows(g, carry):
        acc = bias_vec
        for f in range(NUM_FIELDS):
            pos = lane_off + (g * (LANES * NUM_FIELDS) + f)
            acc = acc + plsc.load_gather(vals_v, [pos])
        out_v[pl.ds(g * LANES, LANES)] = acc
        return carry

    lax.fori_loop(0, RED_STEPS, reduce_rows, 0)

    pltpu.sync_copy(out_v, out_hbm.at[pl.ds(base, B_PER_W)])


def kernel(x, offsets, fc_weight, bias):
    x_flat = x.astype(jnp.int32).reshape(-1)
    offs = offsets.astype(jnp.int32)
    table = fc_weight.reshape(-1)
    out = _fl_kernel(x_flat, offs, table, bias)
    return out.reshape(BATCH, 1)


# speedup vs baseline: 3.3809x; 2.9668x over previous
import functools

import jax
import jax.numpy as jnp
from jax import lax
from jax.experimental import pallas as pl
from jax.experimental.pallas import tpu as pltpu
from jax.experimental.pallas import tpu_sc as plsc

BATCH = 16384
NUM_FIELDS = 26
TOTAL_ROWS = 100000 * 26

NUM_CORES = 2
NUM_SUBCORES = 16
LANES = 16
NUM_WORKERS = NUM_CORES * NUM_SUBCORES

B_PER_W = BATCH // NUM_WORKERS
IDX_PER_W = B_PER_W * NUM_FIELDS
IDX_CHUNKS = IDX_PER_W // LANES
RED_STEPS = B_PER_W // LANES


@functools.partial(
    pl.kernel,
    out_type=jax.ShapeDtypeStruct((BATCH,), jnp.float32),
    mesh=plsc.VectorSubcoreMesh(core_axis_name="c", subcore_axis_name="s"),
    compiler_params=pltpu.CompilerParams(needs_layout_passes=False),
    scratch_types=[
        pltpu.VMEM((IDX_PER_W,), jnp.int32),
        pltpu.VMEM((IDX_PER_W,), jnp.float32),
        pltpu.VMEM((NUM_FIELDS,), jnp.int32),
        pltpu.VMEM((1,), jnp.float32),
        pltpu.VMEM((B_PER_W,), jnp.float32),
        pltpu.SemaphoreType.DMA,
    ],
)
def _fl_kernel(x_hbm, offs_hbm, table_hbm, bias_hbm, out_hbm,
               idx_v, vals_v, offs_v, bias_v, out_v, sem):
    wid = lax.axis_index("s") * NUM_CORES + lax.axis_index("c")
    base = wid * B_PER_W

    pltpu.sync_copy(x_hbm.at[pl.ds(base * NUM_FIELDS, IDX_PER_W)], idx_v)
    pltpu.sync_copy(offs_hbm, offs_v)
    pltpu.sync_copy(bias_hbm, bias_v)

    iota16 = lax.iota(jnp.int32, LANES)

    def add_offsets(j, carry):
        pv = iota16 + j * LANES
        fv = lax.rem(pv, NUM_FIELDS)
        off = plsc.load_gather(offs_v, [fv])
        sl = pl.ds(j * LANES, LANES)
        idx_v[sl] = idx_v[sl] + off
        return carry

    lax.fori_loop(0, IDX_CHUNKS, add_offsets, 0)

    pltpu.async_copy(table_hbm.at[0].at[idx_v], vals_v, sem).wait()

    bias_vec = plsc.load_gather(bias_v, [jnp.zeros((LANES,), jnp.int32)])
    lane_off = iota16 * NUM_FIELDS

    def reduce_rows(g, carry):
        acc = bias_vec
        for f in range(NUM_FIELDS):
            pos = lane_off + (g * (LANES * NUM_FIELDS) + f)
            acc = acc + plsc.load_gather(vals_v, [pos])
        out_v[pl.ds(g * LANES, LANES)] = acc
        return carry

    lax.fori_loop(0, RED_STEPS, reduce_rows, 0)

    pltpu.sync_copy(out_v, out_hbm.at[pl.ds(base, B_PER_W)])


def kernel(x, offsets, fc_weight, bias):
    x_flat = x.astype(jnp.int32).reshape(-1)
    offs = offsets.astype(jnp.int32)
    table = fc_weight.T
    out = _fl_kernel(x_flat, offs, table, bias)
    return out.reshape(BATCH, 1)


# speedup vs baseline: 3.4689x; 1.0260x over previous
import functools

import jax
import jax.numpy as jnp
from jax import lax
from jax.experimental import pallas as pl
from jax.experimental.pallas import tpu as pltpu
from jax.experimental.pallas import tpu_sc as plsc

BATCH = 16384
NUM_FIELDS = 26
TOTAL_ROWS = 100000 * 26

NUM_CORES = 2
NUM_SUBCORES = 16
LANES = 16
NUM_WORKERS = NUM_CORES * NUM_SUBCORES

B_PER_W = BATCH // NUM_WORKERS
IDX_PER_W = B_PER_W * NUM_FIELDS
B_CHUNKS = B_PER_W // LANES


@functools.partial(
    pl.kernel,
    out_type=jax.ShapeDtypeStruct((BATCH,), jnp.float32),
    mesh=plsc.VectorSubcoreMesh(core_axis_name="c", subcore_axis_name="s"),
    compiler_params=pltpu.CompilerParams(needs_layout_passes=False),
    scratch_types=[
        pltpu.VMEM((IDX_PER_W,), jnp.int32),
        pltpu.VMEM((IDX_PER_W,), jnp.int32),
        pltpu.VMEM((IDX_PER_W,), jnp.float32),
        pltpu.VMEM((8 + NUM_FIELDS,), jnp.int32),
        pltpu.VMEM((16,), jnp.float32),
        pltpu.VMEM((B_PER_W,), jnp.float32),
        pltpu.SemaphoreType.DMA,
    ],
)
def _fl_kernel(xt_hbm, offs_hbm, table_hbm, bias_hbm, out_hbm,
               xv, idx_v, vals_v, offs_v, bias_v, out_v, sem):
    wid = lax.axis_index("s") * NUM_CORES + lax.axis_index("c")
    base = wid * B_PER_W

    for f in range(NUM_FIELDS):
        pltpu.sync_copy(
            xt_hbm.at[f, pl.ds(base, B_PER_W)],
            xv.at[pl.ds(f * B_PER_W, B_PER_W)],
        )
    pltpu.sync_copy(offs_hbm, offs_v.at[pl.ds(8, NUM_FIELDS)])
    pltpu.sync_copy(bias_hbm, bias_v.at[pl.ds(8, 1)])

    def add_offsets(j, carry):
        for f in range(NUM_FIELDS):
            off = plsc.load_gather(
                offs_v, [lax.full((LANES,), 8 + f, jnp.int32)]
            )
            sl = pl.ds(f * B_PER_W + j * LANES, LANES)
            idx_v[sl] = xv[sl] + off
        return carry

    lax.fori_loop(0, B_CHUNKS, add_offsets, 0)

    pltpu.async_copy(table_hbm.at[0].at[idx_v], vals_v, sem).wait()

    bias_vec = plsc.load_gather(bias_v, [lax.full((LANES,), 8, jnp.int32)])

    def reduce_rows(g, carry):
        acc = bias_vec
        for f in range(NUM_FIELDS):
            acc = acc + vals_v[pl.ds(f * B_PER_W + g * LANES, LANES)]
        out_v[pl.ds(g * LANES, LANES)] = acc
        return carry

    lax.fori_loop(0, B_CHUNKS, reduce_rows, 0)

    pltpu.sync_copy(out_v, out_hbm.at[pl.ds(base, B_PER_W)])


def kernel(x, offsets, fc_weight, bias):
    xt = x.astype(jnp.int32).T
    offs = offsets.astype(jnp.int32)
    table = fc_weight.T
    out = _fl_kernel(xt, offs, table, bias)
    return out.reshape(BATCH, 1)


# speedup vs baseline: 4.5405x; 1.3089x over previous
import functools

import jax
import jax.numpy as jnp
from jax import lax
from jax.experimental import pallas as pl
from jax.experimental.pallas import tpu as pltpu
from jax.experimental.pallas import tpu_sc as plsc

BATCH = 16384
NUM_FIELDS = 26
TOTAL_ROWS = 100000 * 26

NUM_CORES = 2
NUM_SUBCORES = 16
LANES = 16
NUM_WORKERS = NUM_CORES * NUM_SUBCORES

B_PER_W = BATCH // NUM_WORKERS
IDX_PER_W = B_PER_W * NUM_FIELDS
B_CHUNKS = B_PER_W // LANES


@functools.partial(
    pl.kernel,
    out_type=jax.ShapeDtypeStruct((BATCH,), jnp.float32),
    mesh=plsc.VectorSubcoreMesh(core_axis_name="c", subcore_axis_name="s"),
    compiler_params=pltpu.CompilerParams(needs_layout_passes=False),
    scratch_types=[
        pltpu.VMEM((NUM_FIELDS, B_PER_W), jnp.int32),
        pltpu.VMEM((IDX_PER_W,), jnp.int32),
        pltpu.VMEM((IDX_PER_W,), jnp.float32),
        pltpu.VMEM((8 + NUM_FIELDS,), jnp.int32),
        pltpu.VMEM((16,), jnp.float32),
        pltpu.VMEM((B_PER_W,), jnp.float32),
        pltpu.SemaphoreType.DMA,
    ],
)
def _fl_kernel(xt_hbm, offs_hbm, table_hbm, bias_hbm, out_hbm,
               xv, idx_v, vals_v, offs_v, bias_v, out_v, sem):
    wid = lax.axis_index("s") * NUM_CORES + lax.axis_index("c")
    base = wid * B_PER_W

    pltpu.sync_copy(xt_hbm.at[:, pl.ds(base, B_PER_W)], xv)
    pltpu.sync_copy(offs_hbm, offs_v.at[pl.ds(8, NUM_FIELDS)])
    pltpu.sync_copy(bias_hbm, bias_v.at[pl.ds(8, 1)])

    off_vecs = [
        plsc.load_gather(offs_v, [lax.full((LANES,), 8 + f, jnp.int32)])
        for f in range(NUM_FIELDS)
    ]

    def add_offsets(j, carry):
        sl = pl.ds(j * LANES, LANES)
        for f in range(NUM_FIELDS):
            idx_v[pl.ds(f * B_PER_W + j * LANES, LANES)] = xv[f, sl] + off_vecs[f]
        return carry

    lax.fori_loop(0, B_CHUNKS, add_offsets, 0)

    pltpu.async_copy(table_hbm.at[0].at[idx_v], vals_v, sem).wait()

    bias_vec = plsc.load_gather(bias_v, [lax.full((LANES,), 8, jnp.int32)])

    def reduce_rows(g, carry):
        acc = bias_vec
        for f in range(NUM_FIELDS):
            acc = acc + vals_v[pl.ds(f * B_PER_W + g * LANES, LANES)]
        out_v[pl.ds(g * LANES, LANES)] = acc
        return carry

    lax.fori_loop(0, B_CHUNKS, reduce_rows, 0)

    pltpu.sync_copy(out_v, out_hbm.at[pl.ds(base, B_PER_W)])


def kernel(x, offsets, fc_weight, bias):
    xt = x.astype(jnp.int32).T
    offs = offsets.astype(jnp.int32)
    table = fc_weight.T
    out = _fl_kernel(xt, offs, table, bias)
    return out.reshape(BATCH, 1)


# speedup vs baseline: 4.6961x; 1.0343x over previous
import functools

import jax
import jax.numpy as jnp
from jax import lax
from jax.experimental import pallas as pl
from jax.experimental.pallas import tpu as pltpu
from jax.experimental.pallas import tpu_sc as plsc

BATCH = 16384
NUM_FIELDS = 26
TOTAL_ROWS = 100000 * 26

NUM_CORES = 2
NUM_SUBCORES = 16
LANES = 16
NUM_WORKERS = NUM_CORES * NUM_SUBCORES

B_PER_W = BATCH // NUM_WORKERS
IDX_PER_W = B_PER_W * NUM_FIELDS
B_CHUNKS = B_PER_W // LANES
CHUNK_IDX = NUM_FIELDS * LANES
NUM_SEGS = 4
SEG_CHUNKS = B_CHUNKS // NUM_SEGS
SEG_IDX = SEG_CHUNKS * CHUNK_IDX


@functools.partial(
    pl.kernel,
    out_type=jax.ShapeDtypeStruct((BATCH,), jnp.float32),
    mesh=plsc.VectorSubcoreMesh(core_axis_name="c", subcore_axis_name="s"),
    compiler_params=pltpu.CompilerParams(needs_layout_passes=False),
    scratch_types=[
        pltpu.VMEM((NUM_FIELDS, B_PER_W), jnp.int32),
        pltpu.VMEM((IDX_PER_W,), jnp.int32),
        pltpu.VMEM((IDX_PER_W,), jnp.float32),
        pltpu.VMEM((8 + NUM_FIELDS,), jnp.int32),
        pltpu.VMEM((16,), jnp.float32),
        pltpu.VMEM((B_PER_W,), jnp.float32),
        pltpu.SemaphoreType.DMA,
        pltpu.SemaphoreType.DMA,
        pltpu.SemaphoreType.DMA,
        pltpu.SemaphoreType.DMA,
        pltpu.SemaphoreType.DMA,
    ],
)
def _fl_kernel(xt_hbm, offs_hbm, table_hbm, bias_hbm, out_hbm,
               xv, idx_v, vals_v, offs_v, bias_v, out_v,
               sem_x, sem0, sem1, sem2, sem3):
    wid = lax.axis_index("s") * NUM_CORES + lax.axis_index("c")
    base = wid * B_PER_W
    seg_sems = (sem0, sem1, sem2, sem3)

    x_dma = pltpu.async_copy(xt_hbm.at[:, pl.ds(base, B_PER_W)], xv, sem_x)
    pltpu.sync_copy(offs_hbm, offs_v.at[pl.ds(8, NUM_FIELDS)])
    pltpu.sync_copy(bias_hbm, bias_v.at[pl.ds(8, 1)])

    off_vecs = [
        plsc.load_gather(offs_v, [lax.full((LANES,), 8 + f, jnp.int32)])
        for f in range(NUM_FIELDS)
    ]
    x_dma.wait()

    def build_chunk(c, carry):
        sl = pl.ds(c * LANES, LANES)
        for f in range(NUM_FIELDS):
            idx_v[pl.ds(c * CHUNK_IDX + f * LANES, LANES)] = (
                xv[f, sl] + off_vecs[f]
            )
        return carry

    def fire_segment(s, sem):
        lax.fori_loop(s * SEG_CHUNKS, (s + 1) * SEG_CHUNKS, build_chunk, 0)
        return pltpu.async_copy(
            table_hbm.at[0].at[idx_v.at[pl.ds(s * SEG_IDX, SEG_IDX)]],
            vals_v.at[pl.ds(s * SEG_IDX, SEG_IDX)],
            sem,
        )

    dmas = [fire_segment(s, seg_sems[s]) for s in range(NUM_SEGS)]

    bias_vec = plsc.load_gather(bias_v, [lax.full((LANES,), 8, jnp.int32)])

    def reduce_chunk(c, carry):
        acc = bias_vec
        for f in range(NUM_FIELDS):
            acc = acc + vals_v[pl.ds(c * CHUNK_IDX + f * LANES, LANES)]
        out_v[pl.ds(c * LANES, LANES)] = acc
        return carry

    for s in range(NUM_SEGS):
        dmas[s].wait()
        lax.fori_loop(s * SEG_CHUNKS, (s + 1) * SEG_CHUNKS, reduce_chunk, 0)

    pltpu.sync_copy(out_v, out_hbm.at[pl.ds(base, B_PER_W)])


def kernel(x, offsets, fc_weight, bias):
    xt = x.astype(jnp.int32).T
    offs = offsets.astype(jnp.int32)
    table = fc_weight.T
    out = _fl_kernel(xt, offs, table, bias)
    return out.reshape(BATCH, 1)
